# baseline (speedup 1.0000x reference)
import functools

import jax
import jax.numpy as jnp
from jax import lax
from jax.experimental import pallas as pl
from jax.experimental.pallas import tpu as pltpu

TOKENS = 8192
CHUNK = 1024
NBUF = 4


def _gate_pipelined(x_hbm, w_ref, o_ref, buf, sems):
    nchunks = TOKENS // CHUNK

    def chunk_copy(i, slot):
        return pltpu.make_async_copy(
            x_hbm.at[pl.ds(i * CHUNK, CHUNK), :],
            buf.at[slot],
            sems.at[slot])

    for s in range(NBUF):
        chunk_copy(s, s).start()

    for i in range(nchunks):
        slot = i % NBUF
        chunk_copy(i, slot).wait()
        o_ref[pl.ds(i * CHUNK, CHUNK), :] = lax.dot_general(
            buf[slot], w_ref[...],
            dimension_numbers=(((1,), (1,)), ((), ())),
            preferred_element_type=jnp.float32)
        if i + NBUF < nchunks:
            chunk_copy(i + NBUF, slot).start()


@jax.jit
def kernel(x, W):
    tokens, model_dim = x.shape
    num_experts = W.shape[0]
    return pl.pallas_call(
        _gate_pipelined,
        in_specs=[
            pl.BlockSpec(memory_space=pltpu.MemorySpace.HBM),
            pl.BlockSpec((num_experts, model_dim), lambda: (0, 0)),
        ],
        out_specs=pl.BlockSpec((tokens, num_experts), lambda: (0, 0)),
        out_shape=jax.ShapeDtypeStruct((tokens, num_experts), jnp.float32),
        scratch_shapes=[
            pltpu.VMEM((NBUF, CHUNK, model_dim), jnp.float32),
            pltpu.SemaphoreType.DMA((NBUF,)),
        ],
    )(x, W)


# speedup vs baseline: 1.0139x; 1.0139x over previous
import functools

import jax
import jax.numpy as jnp
from jax import lax
from jax.experimental import pallas as pl
from jax.experimental.pallas import tpu as pltpu

TOKENS = 8192
CHUNK = 256
NBUF = 8


def _gate_pipelined(x_hbm, w_ref, o_ref, buf, sems):
    nchunks = TOKENS // CHUNK

    def chunk_copy(i, slot):
        return pltpu.make_async_copy(
            x_hbm.at[pl.ds(i * CHUNK, CHUNK), :],
            buf.at[slot],
            sems.at[slot])

    for s in range(NBUF):
        chunk_copy(s, s).start()

    for i in range(nchunks):
        slot = i % NBUF
        chunk_copy(i, slot).wait()
        o_ref[pl.ds(i * CHUNK, CHUNK), :] = lax.dot_general(
            buf[slot], w_ref[...],
            dimension_numbers=(((1,), (1,)), ((), ())),
            preferred_element_type=jnp.float32)
        if i + NBUF < nchunks:
            chunk_copy(i + NBUF, slot).start()


@jax.jit
def kernel(x, W):
    tokens, model_dim = x.shape
    num_experts = W.shape[0]
    return pl.pallas_call(
        _gate_pipelined,
        in_specs=[
            pl.BlockSpec(memory_space=pltpu.MemorySpace.HBM),
            pl.BlockSpec((num_experts, model_dim), lambda: (0, 0)),
        ],
        out_specs=pl.BlockSpec((tokens, num_experts), lambda: (0, 0)),
        out_shape=jax.ShapeDtypeStruct((tokens, num_experts), jnp.float32),
        scratch_shapes=[
            pltpu.VMEM((NBUF, CHUNK, model_dim), jnp.float32),
            pltpu.SemaphoreType.DMA((NBUF,)),
        ],
    )(x, W)


# speedup vs baseline: 1.0493x; 1.0349x over previous
import functools

import jax
import jax.numpy as jnp
from jax import lax
from jax.experimental import pallas as pl
from jax.experimental.pallas import tpu as pltpu

TOKEN_BLOCK = 1024


def _gate_block(x0_ref, x1_ref, w_ref, o_ref):
    dims = (((1,), (1,)), ((), ()))
    o_ref[:TOKEN_BLOCK, :] = lax.dot_general(
        x0_ref[...], w_ref[...], dimension_numbers=dims,
        preferred_element_type=jnp.float32)
    o_ref[TOKEN_BLOCK:, :] = lax.dot_general(
        x1_ref[...], w_ref[...], dimension_numbers=dims,
        preferred_element_type=jnp.float32)


@jax.jit
def kernel(x, W):
    tokens, model_dim = x.shape
    num_experts = W.shape[0]
    grid = (tokens // (2 * TOKEN_BLOCK),)
    return pl.pallas_call(
        _gate_block,
        grid=grid,
        in_specs=[
            pl.BlockSpec((TOKEN_BLOCK, model_dim), lambda i: (2 * i, 0)),
            pl.BlockSpec((TOKEN_BLOCK, model_dim), lambda i: (2 * i + 1, 0)),
            pl.BlockSpec((num_experts, model_dim), lambda i: (0, 0)),
        ],
        out_specs=pl.BlockSpec((2 * TOKEN_BLOCK, num_experts),
                               lambda i: (i, 0)),
        out_shape=jax.ShapeDtypeStruct((tokens, num_experts), jnp.float32),
        compiler_params=pltpu.CompilerParams(
            dimension_semantics=("parallel",),
        ),
    )(x, x, W)


# speedup vs baseline: 1.0780x; 1.0273x over previous
import functools

import jax
import jax.numpy as jnp
from jax import lax
from jax.experimental import pallas as pl
from jax.experimental.pallas import tpu as pltpu

TOKEN_BLOCK = 512
XBUFS = 4


def _gate_outer(x_hbm, w_ref, o_hbm):
    tokens, model_dim = x_hbm.shape
    num_experts = w_ref.shape[0]

    def body(x_blk, o_blk):
        o_blk[...] = lax.dot_general(
            x_blk[...], w_ref[...],
            dimension_numbers=(((1,), (1,)), ((), ())),
            preferred_element_type=jnp.float32)

    pipeline = pltpu.emit_pipeline(
        body,
        grid=(tokens // TOKEN_BLOCK,),
        in_specs=[
            pl.BlockSpec((TOKEN_BLOCK, model_dim), lambda i: (i, 0),
                         pipeline_mode=pl.Buffered(buffer_count=XBUFS)),
        ],
        out_specs=[
            pl.BlockSpec((TOKEN_BLOCK, num_experts), lambda i: (i, 0)),
        ],
    )
    pipeline(x_hbm, o_hbm)


@jax.jit
def kernel(x, W):
    tokens, model_dim = x.shape
    num_experts = W.shape[0]
    return pl.pallas_call(
        _gate_outer,
        in_specs=[
            pl.BlockSpec(memory_space=pltpu.MemorySpace.HBM),
            pl.BlockSpec((num_experts, model_dim), lambda: (0, 0)),
        ],
        out_specs=pl.BlockSpec(memory_space=pltpu.MemorySpace.HBM),
        out_shape=jax.ShapeDtypeStruct((tokens, num_experts), jnp.float32),
    )(x, W)
